# baseline (speedup 1.0000x reference)
import functools

import jax
import jax.numpy as jnp
import numpy as np
from jax import lax
from jax.experimental import pallas as pl
from jax.experimental.pallas import tpu as pltpu
from jax.experimental.pallas import tpu_sc as plsc

_EPS = 1e-07
_N_DATA = 100000
_FEAT = 2048
_CDIM = 128
_K = 512
_T = 0.07
_B = 1024

_NC = 2
_NS = 16
_NW = _NC * _NS
_NB = _B // _NW
_CH = 64
_QN = _K // _CH
_STEPS = _NB * _QN


def _embed_body(f_ref, fb_ref, w1_ref, b1_ref, w2_ref, b2_ref, fe_ref, fbe_ref):
    inv_t = 1.0 / _T

    def one(x_ref, w_ref, b_ref, o_ref):
        x = x_ref[...]
        y = lax.dot_general(x, w_ref[...], (((1,), (1,)), ((), ())),
                            preferred_element_type=jnp.float32)
        y = y + b_ref[...]
        nrm = jnp.sqrt(jnp.sum(y * y, axis=1, keepdims=True))
        o_ref[...] = y * (inv_t / nrm)

    one(f_ref, w1_ref, b1_ref, fe_ref)
    one(fb_ref, w2_ref, b2_ref, fbe_ref)


def _embed(f, fb, w1, b1, w2, b2):
    blk = 256
    grid = _B // blk
    return pl.pallas_call(
        _embed_body,
        grid=(grid,),
        in_specs=[
            pl.BlockSpec((blk, _FEAT), lambda i: (i, 0)),
            pl.BlockSpec((blk, _FEAT), lambda i: (i, 0)),
            pl.BlockSpec((_CDIM, _FEAT), lambda i: (0, 0)),
            pl.BlockSpec((1, _CDIM), lambda i: (0, 0)),
            pl.BlockSpec((_CDIM, _FEAT), lambda i: (0, 0)),
            pl.BlockSpec((1, _CDIM), lambda i: (0, 0)),
        ],
        out_specs=[
            pl.BlockSpec((blk, _CDIM), lambda i: (i, 0)),
            pl.BlockSpec((blk, _CDIM), lambda i: (i, 0)),
        ],
        out_shape=[
            jax.ShapeDtypeStruct((_B, _CDIM), jnp.float32),
            jax.ShapeDtypeStruct((_B, _CDIM), jnp.float32),
        ],
    )(f, fb, w1, b1, w2, b2)


def _scr_reduce(scr_ref):
    lanes = lax.iota(jnp.int32, 16)
    tot = plsc.load_gather(scr_ref, [lanes, jnp.zeros((16,), jnp.int32)])
    for c in range(1, 16):
        tot = tot + plsc.load_gather(scr_ref, [lanes, jnp.full((16,), c, jnp.int32)])
    return tot


def _rows16_dot(buf_ref, g, coff, fvecs, scr_ref):
    for u in range(16):
        r = g * 16 + u
        acc = None
        for j in range(4):
            pk = plsc.bitcast(buf_ref[r, pl.ds(coff + 16 * j, 16)], jnp.bfloat16)
            lo, hi = plsc.unpack(pk, format=plsc.PackFormat.INTERLEAVED)
            t = lo * fvecs[2 * j] + hi * fvecs[2 * j + 1]
            acc = t if acc is None else acc + t
        scr_ref[u] = acc
    return _scr_reduce(scr_ref)


def _rows16_dot_perrow(buf_ref, g, coff, f_ref, scr_ref):
    for u in range(16):
        r = g * 16 + u
        acc = None
        for j in range(4):
            pk = plsc.bitcast(buf_ref[r, pl.ds(coff + 16 * j, 16)], jnp.bfloat16)
            lo, hi = plsc.unpack(pk, format=plsc.PackFormat.INTERLEAVED)
            t = (lo * f_ref[r, pl.ds(32 * j, 16)]
                 + hi * f_ref[r, pl.ds(32 * j + 16, 16)])
            acc = t if acc is None else acc + t
        scr_ref[u] = acc
    return _scr_reduce(scr_ref)


def _sc_body(memc, fb1, fb2, idxp, idxn,
             pos1_o, pos2_o, neg1_o, neg2_o,
             f1_v, f2_v, idxp_v, idxn_v, prow_v,
             bufa, bufb,
             neg1_v, neg2_v, pos1_v, pos2_v, scr_v,
             sema, semb, semp):
    wid = lax.axis_index("s") * _NC + lax.axis_index("c")
    base = wid * _NB

    pltpu.sync_copy(fb1.at[pl.ds(base, _NB)], f1_v)
    pltpu.sync_copy(fb2.at[pl.ds(base, _NB)], f2_v)
    pltpu.sync_copy(idxp.at[pl.ds(base, _NB)], idxp_v)
    pltpu.sync_copy(idxn.at[wid], idxn_v)

    def issue(s, buf, sem):
        b = s // _QN
        q = s - b * _QN
        pltpu.async_copy(memc.at[idxn_v.at[b, q]], buf, sem)

    def drain(buf, sem):
        pltpu.make_async_copy(memc.at[pl.ds(0, _CH)], buf, sem).wait()

    pltpu.async_copy(memc.at[idxp_v], prow_v, semp)
    issue(0, bufa, sema)
    pltpu.make_async_copy(memc.at[pl.ds(0, _NB)], prow_v, semp).wait()
    for g in range(_NB // 16):
        pos1_v[pl.ds(16 * g, 16)] = _rows16_dot_perrow(prow_v, g, 0, f1_v, scr_v)
        pos2_v[pl.ds(16 * g, 16)] = _rows16_dot_perrow(prow_v, g, 64, f2_v, scr_v)
    issue(1, bufb, semb)

    def compute(s, buf):
        b = s // _QN
        q = s - b * _QN
        f1vecs = [f1_v[b, pl.ds(16 * j, 16)] for j in range(8)]
        f2vecs = [f2_v[b, pl.ds(16 * j, 16)] for j in range(8)]

        def group(g, _):
            neg1_v[b, pl.ds(q * _CH + 16 * g, 16)] = _rows16_dot(buf, g, 0, f1vecs, scr_v)
            neg2_v[b, pl.ds(q * _CH + 16 * g, 16)] = _rows16_dot(buf, g, 64, f2vecs, scr_v)
            return 0

        lax.fori_loop(0, _CH // 16, group, 0)

    def two_steps(t, _):
        s0 = 2 * t
        s1 = s0 + 1
        drain(bufa, sema)
        compute(s0, bufa)

        @pl.when(s0 + 2 < _STEPS)
        def _():
            issue(s0 + 2, bufa, sema)

        drain(bufb, semb)
        compute(s1, bufb)

        @pl.when(s1 + 2 < _STEPS)
        def _():
            issue(s1 + 2, bufb, semb)

        return 0

    lax.fori_loop(0, _STEPS // 2, two_steps, 0)

    pltpu.sync_copy(pos1_v, pos1_o.at[pl.ds(base, _NB)])
    pltpu.sync_copy(pos2_v, pos2_o.at[pl.ds(base, _NB)])
    pltpu.sync_copy(neg1_v, neg1_o.at[pl.ds(base, _NB)])
    pltpu.sync_copy(neg2_v, neg2_o.at[pl.ds(base, _NB)])


@functools.cache
def _sc_scores_call():
  return functools.partial(
    pl.kernel,
    mesh=plsc.VectorSubcoreMesh(core_axis_name="c", subcore_axis_name="s",
                                num_cores=_NC, num_subcores=_NS),
    compiler_params=pltpu.CompilerParams(needs_layout_passes=False),
    out_type=(
        jax.ShapeDtypeStruct((_B,), jnp.float32),
        jax.ShapeDtypeStruct((_B,), jnp.float32),
        jax.ShapeDtypeStruct((_B, _K), jnp.float32),
        jax.ShapeDtypeStruct((_B, _K), jnp.float32),
    ),
    scratch_types=[
        pltpu.VMEM((_NB, _CDIM), jnp.float32),
        pltpu.VMEM((_NB, _CDIM), jnp.float32),
        pltpu.VMEM((_NB,), jnp.int32),
        pltpu.VMEM((_NB, _QN, _CH), jnp.int32),
        pltpu.VMEM((_NB, _CDIM), jnp.int32),
        pltpu.VMEM((_CH, _CDIM), jnp.int32),
        pltpu.VMEM((_CH, _CDIM), jnp.int32),
        pltpu.VMEM((_NB, _K), jnp.float32),
        pltpu.VMEM((_NB, _K), jnp.float32),
        pltpu.VMEM((_NB,), jnp.float32),
        pltpu.VMEM((_NB,), jnp.float32),
        pltpu.VMEM((16, 16), jnp.float32),
        pltpu.SemaphoreType.DMA,
        pltpu.SemaphoreType.DMA,
        pltpu.SemaphoreType.DMA,
    ],
  )(_sc_body)


def _loss_body(p1_ref, n1_ref, p2_ref, n2_ref, o_ref):
    cn = float(_K) / float(_N_DATA)

    def view(p_ref, n_ref):
        ps = p_ref[...]
        ns = n_ref[...]
        ep = jnp.exp(ps)
        en = jnp.exp(ns)
        s_tot = jnp.sum(ep) + jnp.sum(en)
        z = s_tot / float(_B * (_K + 1)) * float(_N_DATA)
        t_all = (jnp.sum(jnp.log(ep / z + (cn + _EPS)))
                 + jnp.sum(jnp.log(en / z + (cn + _EPS))))
        sig = (jnp.sum(ps) - float(_B) * jnp.log(z)
               + float(_B * _K) * jnp.log(cn) - t_all)
        return -sig / float(_B)

    o_ref[...] = jnp.full((1, 1), view(p1_ref, n1_ref) + view(p2_ref, n2_ref),
                          jnp.float32)


def _loss(p1, n1, p2, n2):
    return pl.pallas_call(
        _loss_body,
        out_shape=jax.ShapeDtypeStruct((1, 1), jnp.float32),
    )(p1, n1, p2, n2)


_PERM = np.concatenate(
    [g * 32 + np.concatenate([np.arange(0, 32, 2), np.arange(1, 32, 2)])
     for g in range(_CDIM // 32)]).astype(np.int32)


def kernel(f, f_info_bank, idx, contrast_idx, W1, b1, W2, b2, memory_v1, memory_v2):
    fe, fbe = _embed(f, f_info_bank, W1, b1.reshape(1, _CDIM), W2, b2.reshape(1, _CDIM))
    idx_neg = contrast_idx[:, 1:].reshape(_NW, _NB, _QN, _CH)
    def pack_bank(m):
        mb = m.astype(jnp.bfloat16).reshape(_N_DATA, _CDIM // 2, 2)
        return lax.bitcast_convert_type(mb, jnp.int32)
    memc = jnp.concatenate([pack_bank(memory_v1), pack_bank(memory_v2)], axis=1)
    pos1, pos2, neg1, neg2 = _sc_scores_call()(
        memc, fbe[:, _PERM], fe[:, _PERM], idx, idx_neg)
    out = _loss(pos1.reshape(8, _CDIM), neg1, pos2.reshape(8, _CDIM), neg2)
    return out[0]


# speedup vs baseline: 2.1274x; 2.1274x over previous
import functools

import jax
import jax.numpy as jnp
import numpy as np
from jax import lax
from jax.experimental import pallas as pl
from jax.experimental.pallas import tpu as pltpu
from jax.experimental.pallas import tpu_sc as plsc

_EPS = 1e-07
_N_DATA = 100000
_FEAT = 2048
_CDIM = 128
_K = 512
_T = 0.07
_B = 1024

_NC = 2
_NS = 16
_NW = _NC * _NS
_NB = _B // _NW
_CH = 64
_QN = _K // _CH
_STEPS = _NB * _QN


def _embed_body(f_ref, fb_ref, w1_ref, b1_ref, w2_ref, b2_ref, fe_ref, fbe_ref):
    inv_t = 1.0 / _T

    def one(x_ref, w_ref, b_ref, o_ref):
        x = x_ref[...]
        y = lax.dot_general(x, w_ref[...], (((1,), (1,)), ((), ())),
                            preferred_element_type=jnp.float32)
        y = y + b_ref[...]
        nrm = jnp.sqrt(jnp.sum(y * y, axis=1, keepdims=True))
        o_ref[...] = y * (inv_t / nrm)

    one(f_ref, w1_ref, b1_ref, fe_ref)
    one(fb_ref, w2_ref, b2_ref, fbe_ref)


def _embed(f, fb, w1, b1, w2, b2):
    blk = 256
    grid = _B // blk
    return pl.pallas_call(
        _embed_body,
        grid=(grid,),
        in_specs=[
            pl.BlockSpec((blk, _FEAT), lambda i: (i, 0)),
            pl.BlockSpec((blk, _FEAT), lambda i: (i, 0)),
            pl.BlockSpec((_CDIM, _FEAT), lambda i: (0, 0)),
            pl.BlockSpec((1, _CDIM), lambda i: (0, 0)),
            pl.BlockSpec((_CDIM, _FEAT), lambda i: (0, 0)),
            pl.BlockSpec((1, _CDIM), lambda i: (0, 0)),
        ],
        out_specs=[
            pl.BlockSpec((blk, _CDIM), lambda i: (i, 0)),
            pl.BlockSpec((blk, _CDIM), lambda i: (i, 0)),
        ],
        out_shape=[
            jax.ShapeDtypeStruct((_B, _CDIM), jnp.float32),
            jax.ShapeDtypeStruct((_B, _CDIM), jnp.float32),
        ],
    )(f, fb, w1, b1, w2, b2)


def _scr_reduce(scr_ref):
    lanes = lax.iota(jnp.int32, 16)
    tot = plsc.load_gather(scr_ref, [lanes, jnp.zeros((16,), jnp.int32)])
    for c in range(1, 16):
        tot = tot + plsc.load_gather(scr_ref, [lanes, jnp.full((16,), c, jnp.int32)])
    return tot


def _rows16_dot(buf_ref, g, coff, fvecs, scr_ref):
    for u in range(16):
        r = g * 16 + u
        acc = None
        for j in range(4):
            pk = plsc.bitcast(buf_ref[r, pl.ds(coff + 16 * j, 16)], jnp.bfloat16)
            lo, hi = plsc.unpack(pk, format=plsc.PackFormat.INTERLEAVED)
            t = lo * fvecs[j] + hi * fvecs[4 + j]
            acc = t if acc is None else acc + t
        scr_ref[u] = acc
    return _scr_reduce(scr_ref)


def _rows16_dot_perrow(buf_ref, g, coff, f_ref, scr_ref):
    for u in range(16):
        r = g * 16 + u
        acc = None
        for j in range(4):
            pk = plsc.bitcast(buf_ref[r, pl.ds(coff + 16 * j, 16)], jnp.bfloat16)
            lo, hi = plsc.unpack(pk, format=plsc.PackFormat.INTERLEAVED)
            t = (lo * f_ref[r, pl.ds(16 * j, 16)]
                 + hi * f_ref[r, pl.ds(64 + 16 * j, 16)])
            acc = t if acc is None else acc + t
        scr_ref[u] = acc
    return _scr_reduce(scr_ref)


def _sc_body(memc, fb1, fb2, idxp, idxn,
             pos1_o, pos2_o, neg1_o, neg2_o,
             f1_v, f2_v, idxp_v, idxn_v, prow_v,
             bufa, bufb, bufc, bufd,
             neg1_v, neg2_v, pos1_v, pos2_v, scr_v,
             sema, semb, semc, semd, semp):
    wid = lax.axis_index("s") * _NC + lax.axis_index("c")
    base = wid * _NB

    pltpu.sync_copy(fb1.at[pl.ds(base, _NB)], f1_v)
    pltpu.sync_copy(fb2.at[pl.ds(base, _NB)], f2_v)
    pltpu.sync_copy(idxp.at[pl.ds(base, _NB)], idxp_v)
    pltpu.sync_copy(idxn.at[wid], idxn_v)

    def issue(s, buf, sem):
        b = s // _QN
        q = s - b * _QN
        pltpu.async_copy(memc.at[idxn_v.at[b, q]], buf, sem)

    def drain(buf, sem):
        pltpu.make_async_copy(memc.at[pl.ds(0, _CH)], buf, sem).wait()

    pltpu.async_copy(memc.at[idxp_v], prow_v, semp)
    issue(0, bufa, sema)
    issue(1, bufb, semb)
    issue(2, bufc, semc)
    pltpu.make_async_copy(memc.at[pl.ds(0, _NB)], prow_v, semp).wait()
    for g in range(_NB // 16):
        pos1_v[pl.ds(16 * g, 16)] = _rows16_dot_perrow(prow_v, g, 0, f1_v, scr_v)
        pos2_v[pl.ds(16 * g, 16)] = _rows16_dot_perrow(prow_v, g, 64, f2_v, scr_v)

    def compute(s, buf):
        b = s // _QN
        q = s - b * _QN
        f1vecs = [f1_v[b, pl.ds(16 * j, 16)] for j in range(8)]
        f2vecs = [f2_v[b, pl.ds(16 * j, 16)] for j in range(8)]

        def group(g, _):
            neg1_v[b, pl.ds(q * _CH + 16 * g, 16)] = _rows16_dot(buf, g, 0, f1vecs, scr_v)
            neg2_v[b, pl.ds(q * _CH + 16 * g, 16)] = _rows16_dot(buf, g, 64, f2vecs, scr_v)
            return 0

        lax.fori_loop(0, _CH // 16, group, 0)

    bufs = (bufa, bufb, bufc, bufd)
    sems = (sema, semb, semc, semd)

    def four_steps(t, _):
        s0 = 4 * t
        for p in range(4):
            s = s0 + p
            drain(bufs[p], sems[p])
            compute(s, bufs[p])
            nxt = (p + 3) % 4

            @pl.when(s + 3 < _STEPS)
            def _():
                issue(s + 3, bufs[nxt], sems[nxt])

        return 0

    lax.fori_loop(0, _STEPS // 4, four_steps, 0)

    pltpu.sync_copy(pos1_v, pos1_o.at[pl.ds(base, _NB)])
    pltpu.sync_copy(pos2_v, pos2_o.at[pl.ds(base, _NB)])
    pltpu.sync_copy(neg1_v, neg1_o.at[pl.ds(base, _NB)])
    pltpu.sync_copy(neg2_v, neg2_o.at[pl.ds(base, _NB)])


@functools.cache
def _sc_scores_call():
  return functools.partial(
    pl.kernel,
    mesh=plsc.VectorSubcoreMesh(core_axis_name="c", subcore_axis_name="s",
                                num_cores=_NC, num_subcores=_NS),
    compiler_params=pltpu.CompilerParams(needs_layout_passes=False),
    out_type=(
        jax.ShapeDtypeStruct((_B,), jnp.float32),
        jax.ShapeDtypeStruct((_B,), jnp.float32),
        jax.ShapeDtypeStruct((_B, _K), jnp.float32),
        jax.ShapeDtypeStruct((_B, _K), jnp.float32),
    ),
    scratch_types=[
        pltpu.VMEM((_NB, _CDIM), jnp.float32),
        pltpu.VMEM((_NB, _CDIM), jnp.float32),
        pltpu.VMEM((_NB,), jnp.int32),
        pltpu.VMEM((_NB, _QN, _CH), jnp.int32),
        pltpu.VMEM((_NB, _CDIM), jnp.int32),
        pltpu.VMEM((_CH, _CDIM), jnp.int32),
        pltpu.VMEM((_CH, _CDIM), jnp.int32),
        pltpu.VMEM((_CH, _CDIM), jnp.int32),
        pltpu.VMEM((_CH, _CDIM), jnp.int32),
        pltpu.VMEM((_NB, _K), jnp.float32),
        pltpu.VMEM((_NB, _K), jnp.float32),
        pltpu.VMEM((_NB,), jnp.float32),
        pltpu.VMEM((_NB,), jnp.float32),
        pltpu.VMEM((16, 16), jnp.float32),
        pltpu.SemaphoreType.DMA,
        pltpu.SemaphoreType.DMA,
        pltpu.SemaphoreType.DMA,
        pltpu.SemaphoreType.DMA,
        pltpu.SemaphoreType.DMA,
    ],
  )(_sc_body)


def _loss_body(p1_ref, n1_ref, p2_ref, n2_ref, o_ref):
    cn = float(_K) / float(_N_DATA)

    def view(p_ref, n_ref):
        ps = p_ref[...]
        ns = n_ref[...]
        ep = jnp.exp(ps)
        en = jnp.exp(ns)
        s_tot = jnp.sum(ep) + jnp.sum(en)
        z = s_tot / float(_B * (_K + 1)) * float(_N_DATA)
        t_all = (jnp.sum(jnp.log(ep / z + (cn + _EPS)))
                 + jnp.sum(jnp.log(en / z + (cn + _EPS))))
        sig = (jnp.sum(ps) - float(_B) * jnp.log(z)
               + float(_B * _K) * jnp.log(cn) - t_all)
        return -sig / float(_B)

    o_ref[...] = jnp.full((1, 1), view(p1_ref, n1_ref) + view(p2_ref, n2_ref),
                          jnp.float32)


def _loss(p1, n1, p2, n2):
    return pl.pallas_call(
        _loss_body,
        out_shape=jax.ShapeDtypeStruct((1, 1), jnp.float32),
    )(p1, n1, p2, n2)


def _pack_body(m1_ref, m2_ref, o_ref):
    def rne_bf16_bits(x):
        u = lax.bitcast_convert_type(x, jnp.int32)
        v = u + (0x7FFF + ((u >> 16) & 1))
        return (v >> 16) & 0xFFFF

    def pack(m_ref):
        lo = rne_bf16_bits(m_ref[:, 0:64])
        hi = rne_bf16_bits(m_ref[:, 64:128])
        return lo | (hi << 16)

    o_ref[...] = jnp.concatenate([pack(m1_ref), pack(m2_ref)], axis=1)


def _pack(m1, m2):
    blk = 2000
    return pl.pallas_call(
        _pack_body,
        grid=(_N_DATA // blk,),
        in_specs=[
            pl.BlockSpec((blk, _CDIM), lambda i: (i, 0)),
            pl.BlockSpec((blk, _CDIM), lambda i: (i, 0)),
        ],
        out_specs=pl.BlockSpec((blk, _CDIM), lambda i: (i, 0)),
        out_shape=jax.ShapeDtypeStruct((_N_DATA, _CDIM), jnp.int32),
    )(m1, m2)


def kernel(f, f_info_bank, idx, contrast_idx, W1, b1, W2, b2, memory_v1, memory_v2):
    memc = _pack(memory_v1, memory_v2)
    fe, fbe = _embed(f, f_info_bank, W1, b1.reshape(1, _CDIM), W2, b2.reshape(1, _CDIM))
    idx_neg = contrast_idx[:, 1:].reshape(_NW, _NB, _QN, _CH)
    pos1, pos2, neg1, neg2 = _sc_scores_call()(
        memc, fbe, fe, idx, idx_neg)
    out = _loss(pos1.reshape(8, _CDIM), neg1, pos2.reshape(8, _CDIM), neg2)
    return out[0]


# speedup vs baseline: 5.8979x; 2.7723x over previous
import functools

import jax
import jax.numpy as jnp
import numpy as np
from jax import lax
from jax.experimental import pallas as pl
from jax.experimental.pallas import tpu as pltpu
from jax.experimental.pallas import tpu_sc as plsc

_EPS = 1e-07
_N_DATA = 100000
_FEAT = 2048
_CDIM = 128
_K = 512
_T = 0.07
_B = 1024

_NC = 2
_NS = 16
_NW = _NC * _NS
_NB = _B // _NW
_CH = 64
_QN = _K // _CH
_STEPS = _NB * _QN


def _embed_body(f_ref, fb_ref, w1_ref, b1_ref, w2_ref, b2_ref, fe_ref, fbe_ref):
    inv_t = 1.0 / _T

    def one(x_ref, w_ref, b_ref, o_ref):
        x = x_ref[...]
        y = lax.dot_general(x, w_ref[...], (((1,), (1,)), ((), ())),
                            preferred_element_type=jnp.float32)
        y = y + b_ref[...]
        nrm = jnp.sqrt(jnp.sum(y * y, axis=1, keepdims=True))
        o_ref[...] = y * (inv_t / nrm)

    one(f_ref, w1_ref, b1_ref, fe_ref)
    one(fb_ref, w2_ref, b2_ref, fbe_ref)


def _embed(f, fb, w1, b1, w2, b2):
    blk = 256
    grid = _B // blk
    return pl.pallas_call(
        _embed_body,
        grid=(grid,),
        in_specs=[
            pl.BlockSpec((blk, _FEAT), lambda i: (i, 0)),
            pl.BlockSpec((blk, _FEAT), lambda i: (i, 0)),
            pl.BlockSpec((_CDIM, _FEAT), lambda i: (0, 0)),
            pl.BlockSpec((1, _CDIM), lambda i: (0, 0)),
            pl.BlockSpec((_CDIM, _FEAT), lambda i: (0, 0)),
            pl.BlockSpec((1, _CDIM), lambda i: (0, 0)),
        ],
        out_specs=[
            pl.BlockSpec((blk, _CDIM), lambda i: (i, 0)),
            pl.BlockSpec((blk, _CDIM), lambda i: (i, 0)),
        ],
        out_shape=[
            jax.ShapeDtypeStruct((_B, _CDIM), jnp.float32),
            jax.ShapeDtypeStruct((_B, _CDIM), jnp.float32),
        ],
    )(f, fb, w1, b1, w2, b2)


_BITREV = (0, 8, 4, 12, 2, 10, 6, 14, 1, 9, 5, 13, 3, 11, 7, 15)


def _make_butterfly():
    lanes = lax.iota(jnp.int32, 16)
    perms = {m: lanes ^ m for m in (8, 4, 2, 1)}
    masks = {m: (lanes & m) == 0 for m in (8, 4, 2, 1)}

    def red(accs):
        cur = [accs[p] for p in _BITREV]
        for m in (8, 4, 2, 1):
            nxt = []
            for k in range(len(cur) // 2):
                a, b = cur[2 * k], cur[2 * k + 1]
                a2 = a + a.at[perms[m]].get(mode="promise_in_bounds")
                b2 = b + b.at[perms[m]].get(mode="promise_in_bounds")
                nxt.append(jnp.where(masks[m], a2, b2))
            cur = nxt
        return cur[0]

    return red


def _rows16_dot(buf_ref, g, coff, fpk, red):
    accs = []
    for u in range(16):
        r = g * 16 + u
        acc = None
        for j in range(4):
            pk = plsc.bitcast(buf_ref[r, pl.ds(coff + 16 * j, 16)], jnp.bfloat16)
            t = pk * fpk[j]
            acc = t if acc is None else acc + t
        lo, hi = plsc.unpack(acc, format=plsc.PackFormat.INTERLEAVED)
        accs.append(lo + hi)
    return red(accs)


def _rows16_dot_perrow(buf_ref, g, coff, f_ref, red):
    accs = []
    for u in range(16):
        r = g * 16 + u
        acc = None
        for j in range(4):
            pk = plsc.bitcast(buf_ref[r, pl.ds(coff + 16 * j, 16)], jnp.bfloat16)
            lo, hi = plsc.unpack(pk, format=plsc.PackFormat.INTERLEAVED)
            t = (lo * f_ref[r, pl.ds(16 * j, 16)]
                 + hi * f_ref[r, pl.ds(64 + 16 * j, 16)])
            acc = t if acc is None else acc + t
        accs.append(acc)
    return red(accs)


def _sc_body(memc, fb1, fb2, idxp, idxn,
             pos1_o, pos2_o, neg1_o, neg2_o,
             f1_v, f2_v, idxp_v, idxn_v, prow_v,
             bufa, bufb, bufc, bufd,
             neg1_v, neg2_v, pos1_v, pos2_v,
             sema, semb, semc, semd, semp):
    wid = lax.axis_index("s") * _NC + lax.axis_index("c")
    base = wid * _NB
    red = _make_butterfly()

    pltpu.sync_copy(fb1.at[pl.ds(base, _NB)], f1_v)
    pltpu.sync_copy(fb2.at[pl.ds(base, _NB)], f2_v)
    pltpu.sync_copy(idxp.at[pl.ds(base, _NB)], idxp_v)
    pltpu.sync_copy(idxn.at[wid], idxn_v)

    def issue(s, buf, sem):
        b = s // _QN
        q = s - b * _QN
        pltpu.async_copy(memc.at[idxn_v.at[b, q]], buf, sem)

    def drain(buf, sem):
        pltpu.make_async_copy(memc.at[pl.ds(0, _CH)], buf, sem).wait()

    pltpu.async_copy(memc.at[idxp_v], prow_v, semp)
    issue(0, bufa, sema)
    issue(1, bufb, semb)
    issue(2, bufc, semc)
    pltpu.make_async_copy(memc.at[pl.ds(0, _NB)], prow_v, semp).wait()
    for g in range(_NB // 16):
        pos1_v[pl.ds(16 * g, 16)] = _rows16_dot_perrow(prow_v, g, 0, f1_v, red)
        pos2_v[pl.ds(16 * g, 16)] = _rows16_dot_perrow(prow_v, g, 64, f2_v, red)

    def compute(s, buf):
        b = s // _QN
        q = s - b * _QN
        f1pk = [plsc.pack(f1_v[b, pl.ds(16 * j, 16)],
                          f1_v[b, pl.ds(64 + 16 * j, 16)],
                          format=plsc.PackFormat.INTERLEAVED) for j in range(4)]
        f2pk = [plsc.pack(f2_v[b, pl.ds(16 * j, 16)],
                          f2_v[b, pl.ds(64 + 16 * j, 16)],
                          format=plsc.PackFormat.INTERLEAVED) for j in range(4)]

        def group(g, _):
            neg1_v[b, pl.ds(q * _CH + 16 * g, 16)] = _rows16_dot(buf, g, 0, f1pk, red)
            neg2_v[b, pl.ds(q * _CH + 16 * g, 16)] = _rows16_dot(buf, g, 64, f2pk, red)
            return 0

        lax.fori_loop(0, _CH // 16, group, 0)

    bufs = (bufa, bufb, bufc, bufd)
    sems = (sema, semb, semc, semd)

    def four_steps(t, _):
        s0 = 4 * t
        for p in range(4):
            s = s0 + p
            drain(bufs[p], sems[p])
            compute(s, bufs[p])
            nxt = (p + 3) % 4

            @pl.when(s + 3 < _STEPS)
            def _():
                issue(s + 3, bufs[nxt], sems[nxt])

        return 0

    lax.fori_loop(0, _STEPS // 4, four_steps, 0)

    pltpu.sync_copy(pos1_v, pos1_o.at[pl.ds(base, _NB)])
    pltpu.sync_copy(pos2_v, pos2_o.at[pl.ds(base, _NB)])
    pltpu.sync_copy(neg1_v, neg1_o.at[pl.ds(base, _NB)])
    pltpu.sync_copy(neg2_v, neg2_o.at[pl.ds(base, _NB)])


@functools.cache
def _sc_scores_call():
  return functools.partial(
    pl.kernel,
    mesh=plsc.VectorSubcoreMesh(core_axis_name="c", subcore_axis_name="s",
                                num_cores=_NC, num_subcores=_NS),
    compiler_params=pltpu.CompilerParams(needs_layout_passes=False),
    out_type=(
        jax.ShapeDtypeStruct((_B,), jnp.float32),
        jax.ShapeDtypeStruct((_B,), jnp.float32),
        jax.ShapeDtypeStruct((_B, _K), jnp.float32),
        jax.ShapeDtypeStruct((_B, _K), jnp.float32),
    ),
    scratch_types=[
        pltpu.VMEM((_NB, _CDIM), jnp.float32),
        pltpu.VMEM((_NB, _CDIM), jnp.float32),
        pltpu.VMEM((_NB,), jnp.int32),
        pltpu.VMEM((_NB, _QN, _CH), jnp.int32),
        pltpu.VMEM((_NB, _CDIM), jnp.int32),
        pltpu.VMEM((_CH, _CDIM), jnp.int32),
        pltpu.VMEM((_CH, _CDIM), jnp.int32),
        pltpu.VMEM((_CH, _CDIM), jnp.int32),
        pltpu.VMEM((_CH, _CDIM), jnp.int32),
        pltpu.VMEM((_NB, _K), jnp.float32),
        pltpu.VMEM((_NB, _K), jnp.float32),
        pltpu.VMEM((_NB,), jnp.float32),
        pltpu.VMEM((_NB,), jnp.float32),
        pltpu.SemaphoreType.DMA,
        pltpu.SemaphoreType.DMA,
        pltpu.SemaphoreType.DMA,
        pltpu.SemaphoreType.DMA,
        pltpu.SemaphoreType.DMA,
    ],
  )(_sc_body)


def _loss_body(p1_ref, n1_ref, p2_ref, n2_ref, o_ref):
    cn = float(_K) / float(_N_DATA)

    def view(p_ref, n_ref):
        ps = p_ref[...]
        ns = n_ref[...]
        ep = jnp.exp(ps)
        en = jnp.exp(ns)
        s_tot = jnp.sum(ep) + jnp.sum(en)
        z = s_tot / float(_B * (_K + 1)) * float(_N_DATA)
        t_all = (jnp.sum(jnp.log(ep / z + (cn + _EPS)))
                 + jnp.sum(jnp.log(en / z + (cn + _EPS))))
        sig = (jnp.sum(ps) - float(_B) * jnp.log(z)
               + float(_B * _K) * jnp.log(cn) - t_all)
        return -sig / float(_B)

    o_ref[...] = jnp.full((1, 1), view(p1_ref, n1_ref) + view(p2_ref, n2_ref),
                          jnp.float32)


def _loss(p1, n1, p2, n2):
    return pl.pallas_call(
        _loss_body,
        out_shape=jax.ShapeDtypeStruct((1, 1), jnp.float32),
    )(p1, n1, p2, n2)


def _pack_body(m1_ref, m2_ref, o_ref):
    def rne_bf16_bits(x):
        u = lax.bitcast_convert_type(x, jnp.int32)
        v = u + (0x7FFF + ((u >> 16) & 1))
        return (v >> 16) & 0xFFFF

    def pack(m_ref):
        lo = rne_bf16_bits(m_ref[:, 0:64])
        hi = rne_bf16_bits(m_ref[:, 64:128])
        return lo | (hi << 16)

    o_ref[...] = jnp.concatenate([pack(m1_ref), pack(m2_ref)], axis=1)


def _pack(m1, m2):
    blk = 2000
    return pl.pallas_call(
        _pack_body,
        grid=(_N_DATA // blk,),
        in_specs=[
            pl.BlockSpec((blk, _CDIM), lambda i: (i, 0)),
            pl.BlockSpec((blk, _CDIM), lambda i: (i, 0)),
        ],
        out_specs=pl.BlockSpec((blk, _CDIM), lambda i: (i, 0)),
        out_shape=jax.ShapeDtypeStruct((_N_DATA, _CDIM), jnp.int32),
    )(m1, m2)


def kernel(f, f_info_bank, idx, contrast_idx, W1, b1, W2, b2, memory_v1, memory_v2):
    memc = _pack(memory_v1, memory_v2)
    fe, fbe = _embed(f, f_info_bank, W1, b1.reshape(1, _CDIM), W2, b2.reshape(1, _CDIM))
    idx_neg = contrast_idx[:, 1:].reshape(_NW, _NB, _QN, _CH)
    pos1, pos2, neg1, neg2 = _sc_scores_call()(
        memc, fbe, fe, idx, idx_neg)
    out = _loss(pos1.reshape(8, _CDIM), neg1, pos2.reshape(8, _CDIM), neg2)
    return out[0]


# speedup vs baseline: 6.3651x; 1.0792x over previous
import functools

import jax
import jax.numpy as jnp
from jax import lax
from jax.experimental import pallas as pl
from jax.experimental.pallas import tpu as pltpu
from jax.experimental.pallas import tpu_sc as plsc

_EPS = 1e-07
_N_DATA = 100000
_FEAT = 2048
_CDIM = 128
_K = 512
_T = 0.07
_B = 1024

_NC = 2
_NS = 16
_NW = _NC * _NS
_NB = _B // _NW
_CH = 64
_QN = _K // _CH
_STEPS = _NB * _QN


def _embed_body(f_ref, fb_ref, w1_ref, b1_ref, w2_ref, b2_ref, fe_ref, fbe_ref):
    inv_t = 1.0 / _T

    def one(x_ref, w_ref, b_ref, o_ref):
        x = x_ref[...]
        y = lax.dot_general(x, w_ref[...], (((1,), (1,)), ((), ())),
                            preferred_element_type=jnp.float32)
        y = y + b_ref[...]
        nrm = jnp.sqrt(jnp.sum(y * y, axis=1, keepdims=True))
        o_ref[...] = y * (inv_t / nrm)

    one(f_ref, w1_ref, b1_ref, fe_ref)
    one(fb_ref, w2_ref, b2_ref, fbe_ref)


def _embed(f, fb, w1, b1, w2, b2):
    blk = 512
    grid = _B // blk
    return pl.pallas_call(
        _embed_body,
        grid=(grid,),
        in_specs=[
            pl.BlockSpec((blk, _FEAT), lambda i: (i, 0)),
            pl.BlockSpec((blk, _FEAT), lambda i: (i, 0)),
            pl.BlockSpec((_CDIM, _FEAT), lambda i: (0, 0)),
            pl.BlockSpec((1, _CDIM), lambda i: (0, 0)),
            pl.BlockSpec((_CDIM, _FEAT), lambda i: (0, 0)),
            pl.BlockSpec((1, _CDIM), lambda i: (0, 0)),
        ],
        out_specs=[
            pl.BlockSpec((blk, _CDIM), lambda i: (i, 0)),
            pl.BlockSpec((blk, _CDIM), lambda i: (i, 0)),
        ],
        out_shape=[
            jax.ShapeDtypeStruct((_B, _CDIM), jnp.float32),
            jax.ShapeDtypeStruct((_B, _CDIM), jnp.float32),
        ],
    )(f, fb, w1, b1, w2, b2)


_BITREV = (0, 8, 4, 12, 2, 10, 6, 14, 1, 9, 5, 13, 3, 11, 7, 15)


def _make_butterfly():
    lanes = lax.iota(jnp.int32, 16)
    perms = {m: lanes ^ m for m in (8, 4, 2, 1)}
    masks = {m: (lanes & m) == 0 for m in (8, 4, 2, 1)}

    def red(accs):
        cur = [accs[p] for p in _BITREV]
        for m in (8, 4, 2, 1):
            nxt = []
            for k in range(len(cur) // 2):
                a, b = cur[2 * k], cur[2 * k + 1]
                a2 = a + a.at[perms[m]].get(mode="promise_in_bounds")
                b2 = b + b.at[perms[m]].get(mode="promise_in_bounds")
                nxt.append(jnp.where(masks[m], a2, b2))
            cur = nxt
        return cur[0]

    return red


def _rows16_dot(buf_ref, g, coff, fpk, red):
    accs = []
    for u in range(16):
        r = g * 16 + u
        acc = None
        for j in range(4):
            pk = plsc.bitcast(buf_ref[r, pl.ds(coff + 16 * j, 16)], jnp.bfloat16)
            t = pk * fpk[j]
            acc = t if acc is None else acc + t
        lo, hi = plsc.unpack(acc, format=plsc.PackFormat.INTERLEAVED)
        accs.append(lo + hi)
    return red(accs)


def _rows16_dot_perrow(buf_ref, g, coff, f_ref, red):
    accs = []
    for u in range(16):
        r = g * 16 + u
        acc = None
        for j in range(4):
            pk = plsc.bitcast(buf_ref[r, pl.ds(coff + 16 * j, 16)], jnp.bfloat16)
            lo, hi = plsc.unpack(pk, format=plsc.PackFormat.INTERLEAVED)
            t = (lo * f_ref[r, pl.ds(16 * j, 16)]
                 + hi * f_ref[r, pl.ds(64 + 16 * j, 16)])
            acc = t if acc is None else acc + t
        accs.append(acc)
    return red(accs)


def _sc_body(memc, fb1, fb2, idxp, idxn,
             pos1_o, pos2_o, neg1_o, neg2_o,
             f1_v, f2_v, idxp_v, idxn_v, prow_v,
             bufa, bufb, bufc, bufd,
             neg1_v, neg2_v, pos1_v, pos2_v,
             sema, semb, semc, semd, semp):
    wid = lax.axis_index("s") * _NC + lax.axis_index("c")
    base = wid * _NB
    red = _make_butterfly()

    pltpu.sync_copy(fb1.at[pl.ds(base, _NB)], f1_v)
    pltpu.sync_copy(fb2.at[pl.ds(base, _NB)], f2_v)
    pltpu.sync_copy(idxp.at[pl.ds(base, _NB)], idxp_v)
    pltpu.sync_copy(idxn.at[wid], idxn_v)

    def issue(s, buf, sem):
        b = s // _QN
        q = s - b * _QN
        pltpu.async_copy(memc.at[idxn_v.at[b, q]], buf, sem)

    def drain(buf, sem):
        pltpu.make_async_copy(memc.at[pl.ds(0, _CH)], buf, sem).wait()

    pltpu.async_copy(memc.at[idxp_v], prow_v, semp)
    issue(0, bufa, sema)
    issue(1, bufb, semb)
    issue(2, bufc, semc)
    pltpu.make_async_copy(memc.at[pl.ds(0, _NB)], prow_v, semp).wait()
    for g in range(_NB // 16):
        pos1_v[pl.ds(16 * g, 16)] = _rows16_dot_perrow(prow_v, g, 0, f1_v, red)
        pos2_v[pl.ds(16 * g, 16)] = _rows16_dot_perrow(prow_v, g, 64, f2_v, red)

    def compute(s, buf):
        b = s // _QN
        q = s - b * _QN
        f1pk = [plsc.pack(f1_v[b, pl.ds(16 * j, 16)],
                          f1_v[b, pl.ds(64 + 16 * j, 16)],
                          format=plsc.PackFormat.INTERLEAVED) for j in range(4)]
        f2pk = [plsc.pack(f2_v[b, pl.ds(16 * j, 16)],
                          f2_v[b, pl.ds(64 + 16 * j, 16)],
                          format=plsc.PackFormat.INTERLEAVED) for j in range(4)]

        def group(g, _):
            neg1_v[b, pl.ds(q * _CH + 16 * g, 16)] = _rows16_dot(buf, g, 0, f1pk, red)
            neg2_v[b, pl.ds(q * _CH + 16 * g, 16)] = _rows16_dot(buf, g, 64, f2pk, red)
            return 0

        lax.fori_loop(0, _CH // 16, group, 0)

    bufs = (bufa, bufb, bufc, bufd)
    sems = (sema, semb, semc, semd)

    def four_steps(t, _):
        s0 = 4 * t
        for p in range(4):
            s = s0 + p
            drain(bufs[p], sems[p])
            compute(s, bufs[p])
            nxt = (p + 3) % 4

            @pl.when(s + 3 < _STEPS)
            def _():
                issue(s + 3, bufs[nxt], sems[nxt])

        return 0

    lax.fori_loop(0, _STEPS // 4, four_steps, 0)

    pltpu.sync_copy(pos1_v, pos1_o.at[pl.ds(base, _NB)])
    pltpu.sync_copy(pos2_v, pos2_o.at[pl.ds(base, _NB)])
    pltpu.sync_copy(neg1_v, neg1_o.at[pl.ds(base, _NB)])
    pltpu.sync_copy(neg2_v, neg2_o.at[pl.ds(base, _NB)])


@functools.cache
def _sc_scores_call():
  return functools.partial(
    pl.kernel,
    mesh=plsc.VectorSubcoreMesh(core_axis_name="c", subcore_axis_name="s",
                                num_cores=_NC, num_subcores=_NS),
    compiler_params=pltpu.CompilerParams(needs_layout_passes=False),
    out_type=(
        jax.ShapeDtypeStruct((_B,), jnp.float32),
        jax.ShapeDtypeStruct((_B,), jnp.float32),
        jax.ShapeDtypeStruct((_B, _K), jnp.float32),
        jax.ShapeDtypeStruct((_B, _K), jnp.float32),
    ),
    scratch_types=[
        pltpu.VMEM((_NB, _CDIM), jnp.float32),
        pltpu.VMEM((_NB, _CDIM), jnp.float32),
        pltpu.VMEM((_NB,), jnp.int32),
        pltpu.VMEM((_NB, _QN, _CH), jnp.int32),
        pltpu.VMEM((_NB, _CDIM), jnp.int32),
        pltpu.VMEM((_CH, _CDIM), jnp.int32),
        pltpu.VMEM((_CH, _CDIM), jnp.int32),
        pltpu.VMEM((_CH, _CDIM), jnp.int32),
        pltpu.VMEM((_CH, _CDIM), jnp.int32),
        pltpu.VMEM((_NB, _K), jnp.float32),
        pltpu.VMEM((_NB, _K), jnp.float32),
        pltpu.VMEM((_NB,), jnp.float32),
        pltpu.VMEM((_NB,), jnp.float32),
        pltpu.SemaphoreType.DMA,
        pltpu.SemaphoreType.DMA,
        pltpu.SemaphoreType.DMA,
        pltpu.SemaphoreType.DMA,
        pltpu.SemaphoreType.DMA,
    ],
  )(_sc_body)


def _loss_body(p1_ref, n1_ref, p2_ref, n2_ref, o_ref):
    cn = float(_K) / float(_N_DATA)

    def view(p_ref, n_ref):
        ps = p_ref[...]
        ns = n_ref[...]
        ep = jnp.exp(ps)
        en = jnp.exp(ns)
        s_tot = jnp.sum(ep) + jnp.sum(en)
        z = s_tot / float(_B * (_K + 1)) * float(_N_DATA)
        t_all = (jnp.sum(jnp.log(ep / z + (cn + _EPS)))
                 + jnp.sum(jnp.log(en / z + (cn + _EPS))))
        sig = (jnp.sum(ps) - float(_B) * jnp.log(z)
               + float(_B * _K) * jnp.log(cn) - t_all)
        return -sig / float(_B)

    o_ref[...] = jnp.full((1, 1), view(p1_ref, n1_ref) + view(p2_ref, n2_ref),
                          jnp.float32)


def _loss(p1, n1, p2, n2):
    return pl.pallas_call(
        _loss_body,
        out_shape=jax.ShapeDtypeStruct((1, 1), jnp.float32),
    )(p1, n1, p2, n2)


def _pack_body(m1_ref, m2_ref, o_ref):
    def rne_bf16_bits(x):
        u = lax.bitcast_convert_type(x, jnp.int32)
        v = u + (0x7FFF + ((u >> 16) & 1))
        return (v >> 16) & 0xFFFF

    def pack(m_ref):
        lo = rne_bf16_bits(m_ref[:, 0:64])
        hi = rne_bf16_bits(m_ref[:, 64:128])
        return lo | (hi << 16)

    o_ref[...] = jnp.concatenate([pack(m1_ref), pack(m2_ref)], axis=1)


def _pack(m1, m2):
    blk = 10000
    return pl.pallas_call(
        _pack_body,
        grid=(_N_DATA // blk,),
        in_specs=[
            pl.BlockSpec((blk, _CDIM), lambda i: (i, 0)),
            pl.BlockSpec((blk, _CDIM), lambda i: (i, 0)),
        ],
        out_specs=pl.BlockSpec((blk, _CDIM), lambda i: (i, 0)),
        out_shape=jax.ShapeDtypeStruct((_N_DATA, _CDIM), jnp.int32),
    )(m1, m2)


def kernel(f, f_info_bank, idx, contrast_idx, W1, b1, W2, b2, memory_v1, memory_v2):
    memc = _pack(memory_v1, memory_v2)
    fe, fbe = _embed(f, f_info_bank, W1, b1.reshape(1, _CDIM), W2, b2.reshape(1, _CDIM))
    idx_neg = contrast_idx[:, 1:].reshape(_NW, _NB, _QN, _CH)
    pos1, pos2, neg1, neg2 = _sc_scores_call()(
        memc, fbe, fe, idx, idx_neg)
    out = _loss(pos1.reshape(8, _CDIM), neg1, pos2.reshape(8, _CDIM), neg2)
    return out[0]
